# Initial kernel scaffold; baseline (speedup 1.0000x reference)
import jax
import jax.numpy as jnp
from jax.experimental import pallas as pl


def kernel(x, adj):
    raise NotImplementedError("write your pallas kernel here")


# speedup vs baseline: 1.3538x; 1.3538x over previous
import jax
import jax.numpy as jnp
from jax.experimental import pallas as pl
from jax.experimental.pallas import tpu as pltpu

_ALPHA = 0.1
_K = 10
_TILE = 256


def _appnp_body(x_ref, adj_ref, o_ref, hf_ref, hb_ref):
    n = x_ref.shape[0]
    num_tiles = n // _TILE
    hf_ref[...] = x_ref[...]

    def step(_, carry):
        hb_ref[...] = hf_ref[...].astype(jnp.bfloat16)

        def tile_body(i, c):
            sl = pl.ds(i * _TILE, _TILE)
            ah = jnp.dot(adj_ref[sl, :], hb_ref[...],
                         preferred_element_type=jnp.float32)
            hf_ref[sl, :] = (1.0 - _ALPHA) * ah + _ALPHA * x_ref[sl, :]
            return c

        return jax.lax.fori_loop(0, num_tiles, tile_body, carry)

    jax.lax.fori_loop(0, _K, step, 0)
    o_ref[...] = hf_ref[...]


def kernel(x, adj):
    adj_bf16 = adj.astype(jnp.bfloat16)
    n, f = x.shape
    return pl.pallas_call(
        _appnp_body,
        out_shape=jax.ShapeDtypeStruct(x.shape, x.dtype),
        scratch_shapes=[
            pltpu.VMEM((n, f), jnp.float32),
            pltpu.VMEM((n, f), jnp.bfloat16),
        ],
        compiler_params=pltpu.CompilerParams(
            vmem_limit_bytes=64 * 1024 * 1024,
        ),
    )(x, adj_bf16)


# speedup vs baseline: 1.5675x; 1.1578x over previous
import jax
import jax.numpy as jnp
from jax.experimental import pallas as pl
from jax.experimental.pallas import tpu as pltpu

_ALPHA = 0.1
_K = 10
_TILE = 256


def _appnp_body(x_ref, adj_win_ref, o_ref, ab_ref, hf_ref, hb_ref):
    i = pl.program_id(0)
    n = x_ref.shape[0]
    num_tiles = n // _TILE
    sl = pl.ds(i * _TILE, _TILE)

    @pl.when(i == 0)
    def _init():
        hb_ref[...] = x_ref[...].astype(jnp.bfloat16)

    ab_ref[sl, :] = adj_win_ref[...].astype(jnp.bfloat16)
    ah0 = jnp.dot(ab_ref[sl, :], hb_ref[...],
                  preferred_element_type=jnp.float32)
    hf_ref[sl, :] = (1.0 - _ALPHA) * ah0 + _ALPHA * x_ref[sl, :]

    @pl.when(i == num_tiles - 1)
    def _tail():
        def step(_, carry):
            hb_ref[...] = hf_ref[...].astype(jnp.bfloat16)

            def tile_body(t, c):
                tsl = pl.ds(t * _TILE, _TILE)
                ah = jnp.dot(ab_ref[tsl, :], hb_ref[...],
                             preferred_element_type=jnp.float32)
                hf_ref[tsl, :] = (1.0 - _ALPHA) * ah + _ALPHA * x_ref[tsl, :]
                return c

            return jax.lax.fori_loop(0, num_tiles, tile_body, carry)

        jax.lax.fori_loop(0, _K - 1, step, 0)
        o_ref[...] = hf_ref[...]


def kernel(x, adj):
    n, f = x.shape
    num_tiles = n // _TILE
    return pl.pallas_call(
        _appnp_body,
        grid=(num_tiles,),
        in_specs=[
            pl.BlockSpec((n, f), lambda i: (0, 0)),
            pl.BlockSpec((_TILE, n), lambda i: (i, 0)),
        ],
        out_specs=pl.BlockSpec((n, f), lambda i: (0, 0)),
        out_shape=jax.ShapeDtypeStruct(x.shape, x.dtype),
        scratch_shapes=[
            pltpu.VMEM((n, n), jnp.bfloat16),
            pltpu.VMEM((n, f), jnp.float32),
            pltpu.VMEM((n, f), jnp.bfloat16),
        ],
        compiler_params=pltpu.CompilerParams(
            vmem_limit_bytes=64 * 1024 * 1024,
        ),
    )(x, adj)


# speedup vs baseline: 1.9778x; 1.2617x over previous
import jax
import jax.numpy as jnp
from jax.experimental import pallas as pl
from jax.experimental.pallas import tpu as pltpu

_ALPHA = 0.1
_K = 10
_TILE = 512


def _appnp_body(x_ref, adj_win_ref, o_ref, ab_ref, hf_ref, hb_ref):
    i = pl.program_id(0)
    n = x_ref.shape[0]
    num_tiles = n // _TILE
    sl = pl.ds(i * _TILE, _TILE)

    @pl.when(i == 0)
    def _init():
        hb_ref[...] = x_ref[...].astype(jnp.bfloat16)

    ab_ref[sl, :] = adj_win_ref[...].astype(jnp.bfloat16)
    ah0 = jnp.dot(ab_ref[sl, :], hb_ref[...],
                  preferred_element_type=jnp.float32)
    hf_ref[sl, :] = (1.0 - _ALPHA) * ah0 + _ALPHA * x_ref[sl, :]

    @pl.when(i == num_tiles - 1)
    def _tail():
        def step(_, carry):
            hb_ref[...] = hf_ref[...].astype(jnp.bfloat16)
            for t in range(num_tiles):
                tsl = pl.ds(t * _TILE, _TILE)
                ah = jnp.dot(ab_ref[tsl, :], hb_ref[...],
                             preferred_element_type=jnp.float32)
                hf_ref[tsl, :] = (1.0 - _ALPHA) * ah + _ALPHA * x_ref[tsl, :]
            return carry

        jax.lax.fori_loop(0, _K - 1, step, 0)
        o_ref[...] = hf_ref[...]


def kernel(x, adj):
    n, f = x.shape
    num_tiles = n // _TILE
    return pl.pallas_call(
        _appnp_body,
        grid=(num_tiles,),
        in_specs=[
            pl.BlockSpec((n, f), lambda i: (0, 0)),
            pl.BlockSpec((_TILE, n), lambda i: (i, 0)),
        ],
        out_specs=pl.BlockSpec((n, f), lambda i: (0, 0)),
        out_shape=jax.ShapeDtypeStruct(x.shape, x.dtype),
        scratch_shapes=[
            pltpu.VMEM((n, n), jnp.bfloat16),
            pltpu.VMEM((n, f), jnp.float32),
            pltpu.VMEM((n, f), jnp.bfloat16),
        ],
        compiler_params=pltpu.CompilerParams(
            vmem_limit_bytes=64 * 1024 * 1024,
        ),
    )(x, adj)
